# baseline (device time: 686219 ns/iter reference)
import jax
import jax.numpy as jnp
from jax import lax
from jax.experimental import pallas as pl
from jax.experimental.pallas import tpu as pltpu

N_DEV = 4
N_TOK = 2048
D_MODEL = 512
D_HID = 1024
N_EXP = 16
CAP = 102
EXP_PER_DEV = N_EXP // N_DEV
SLOTS = EXP_PER_DEV * CAP


def _moe_gemm_allgather(x_disp, expert_W):

    def body(xd_ref, w_ref, out_ref, send_sems, recv_sems):
        my = lax.axis_index("i")
        left = (my - 1) % N_DEV
        right = (my + 1) % N_DEV

        barrier = pltpu.get_barrier_semaphore()
        for nbr in (left, right):
            pl.semaphore_signal(
                barrier, inc=1,
                device_id=(nbr,), device_id_type=pl.DeviceIdType.MESH,
            )
        pl.semaphore_wait(barrier, 2)

        for k in range(EXP_PER_DEV):
            out_ref[pl.ds(my * EXP_PER_DEV + k, 1)] = jnp.dot(
                xd_ref[k], w_ref[k], preferred_element_type=jnp.float32
            )[None]

        for h in range(N_DEV - 1):
            src = (my - h) % N_DEV
            blk = out_ref.at[pl.ds(src * EXP_PER_DEV, EXP_PER_DEV)]
            rdma = pltpu.make_async_remote_copy(
                src_ref=blk,
                dst_ref=blk,
                send_sem=send_sems.at[h],
                recv_sem=recv_sems.at[h],
                device_id=(right,),
                device_id_type=pl.DeviceIdType.MESH,
            )
            rdma.start()
            rdma.wait()

    return pl.pallas_call(
        body,
        out_shape=jax.ShapeDtypeStruct((N_EXP, CAP, D_HID), jnp.float32),
        in_specs=[
            pl.BlockSpec(memory_space=pltpu.VMEM),
            pl.BlockSpec(memory_space=pltpu.VMEM),
        ],
        out_specs=pl.BlockSpec(memory_space=pltpu.VMEM),
        scratch_shapes=[
            pltpu.SemaphoreType.DMA((N_DEV - 1,)),
            pltpu.SemaphoreType.DMA((N_DEV - 1,)),
        ],
        compiler_params=pltpu.CompilerParams(collective_id=0),
    )(x_disp, expert_W)


def kernel(x, router_W, route_idx, expert_W):
    del router_W
    my = lax.axis_index("i")

    e_tok = route_idx[:, 0].astype(jnp.int32)
    onehot = e_tok[:, None] == jnp.arange(N_EXP, dtype=jnp.int32)[None, :]
    pos = jnp.cumsum(onehot.astype(jnp.int32), axis=0) - 1
    pos_tok = jnp.take_along_axis(pos, e_tok[:, None], axis=1)[:, 0]
    keep = pos_tok < CAP
    gslot = jnp.where(keep, e_tok * CAP + pos_tok, N_EXP * CAP)

    base = my * SLOTS
    in_my = (gslot >= base) & (gslot < base + SLOTS)
    lslot = jnp.where(in_my, gslot - base, SLOTS)
    x_disp = jnp.zeros((SLOTS, D_MODEL), x.dtype).at[lslot].set(x, mode="drop")

    y_all = _moe_gemm_allgather(
        x_disp.reshape(EXP_PER_DEV, CAP, D_MODEL), expert_W
    ).reshape(N_EXP * CAP, D_HID)

    y_pad = jnp.concatenate([y_all, jnp.zeros((1, D_HID), jnp.float32)], axis=0)
    return jnp.take(y_pad, gslot, axis=0)


# device time: 93326 ns/iter; 7.3529x vs baseline; 7.3529x over previous
import jax
import jax.numpy as jnp
from jax import lax
from jax.experimental import pallas as pl
from jax.experimental.pallas import tpu as pltpu

N_DEV = 4
N_TOK = 2048
D_MODEL = 512
D_HID = 1024
N_EXP = 16
CAP = 102
EXP_PER_DEV = N_EXP // N_DEV
SLOTS = EXP_PER_DEV * CAP


def _moe_pallas(x, gslot_row, gslot_col, expert_W):
    def body(x_ref, gsr_ref, gsc_ref, w_ref, out_ref, ygath_ref,
             send_sems, recv_sems):
        my = lax.axis_index("i")
        left = (my - 1) % N_DEV
        right = (my + 1) % N_DEV

        barrier = pltpu.get_barrier_semaphore()
        for nbr in (left, right):
            pl.semaphore_signal(
                barrier, inc=1,
                device_id=(nbr,), device_id_type=pl.DeviceIdType.MESH,
            )
        pl.semaphore_wait(barrier, 2)

        gsr = gsr_ref[0, :]
        xv = x_ref[:, :]

        for k in range(EXP_PER_DEV):
            base = (my * EXP_PER_DEV + k) * CAP
            slot_iota = lax.broadcasted_iota(jnp.int32, (CAP, N_TOK), 0)
            sel = (gsr[None, :] == slot_iota + base).astype(jnp.float32)
            xk = jnp.dot(sel, xv, preferred_element_type=jnp.float32)
            yk = jnp.dot(xk, w_ref[k], preferred_element_type=jnp.float32)
            ygath_ref[pl.ds(my * EXP_PER_DEV + k, 1)] = yk[None]

        gsc = gsc_ref[:, :]

        def combine(origin):
            for j in range(EXP_PER_DEV):
                e = origin * EXP_PER_DEV + j
                cap_iota = lax.broadcasted_iota(jnp.int32, (N_TOK, CAP), 1)
                p = (gsc == cap_iota + e * CAP).astype(jnp.bfloat16)
                ye = ygath_ref[pl.ds(e, 1)][0].astype(jnp.bfloat16)
                out_ref[:, :] += jnp.dot(p, ye, preferred_element_type=jnp.float32)

        out_ref[:, :] = jnp.zeros((N_TOK, D_HID), jnp.float32)

        prev = None
        for h in range(N_DEV - 1):
            src = (my - h) % N_DEV
            rdma = pltpu.make_async_remote_copy(
                src_ref=ygath_ref.at[pl.ds(src * EXP_PER_DEV, EXP_PER_DEV)],
                dst_ref=ygath_ref.at[pl.ds(src * EXP_PER_DEV, EXP_PER_DEV)],
                send_sem=send_sems.at[h],
                recv_sem=recv_sems.at[h],
                device_id=(right,),
                device_id_type=pl.DeviceIdType.MESH,
            )
            rdma.start()
            if prev is not None:
                prev.wait_send()
            combine(src)
            rdma.wait_recv()
            prev = rdma
        prev.wait_send()
        combine((my - N_DEV + 1) % N_DEV)

    return pl.pallas_call(
        body,
        out_shape=jax.ShapeDtypeStruct((N_TOK, D_HID), jnp.float32),
        in_specs=[pl.BlockSpec(memory_space=pltpu.VMEM)] * 4,
        out_specs=pl.BlockSpec(memory_space=pltpu.VMEM),
        scratch_shapes=[
            pltpu.VMEM((N_EXP, CAP, D_HID), jnp.float32),
            pltpu.SemaphoreType.DMA((N_DEV - 1,)),
            pltpu.SemaphoreType.DMA((N_DEV - 1,)),
        ],
        compiler_params=pltpu.CompilerParams(collective_id=0),
    )(x, gslot_row, gslot_col, expert_W)


def kernel(x, router_W, route_idx, expert_W):
    del router_W

    e_tok = route_idx[:, 0].astype(jnp.int32)
    onehot = e_tok[:, None] == jnp.arange(N_EXP, dtype=jnp.int32)[None, :]
    pos = jnp.cumsum(onehot.astype(jnp.int32), axis=0) - 1
    pos_tok = jnp.take_along_axis(pos, e_tok[:, None], axis=1)[:, 0]
    keep = pos_tok < CAP
    gslot = jnp.where(keep, e_tok * CAP + pos_tok, N_EXP * CAP)

    return _moe_pallas(
        x, gslot.reshape(1, N_TOK), gslot.reshape(N_TOK, 1), expert_W
    )


# device time: 58679 ns/iter; 11.6945x vs baseline; 1.5904x over previous
import jax
import jax.numpy as jnp
from jax import lax
from jax.experimental import pallas as pl
from jax.experimental.pallas import tpu as pltpu

N_DEV = 4
N_TOK = 2048
D_MODEL = 512
D_HID = 1024
N_EXP = 16
CAP = 102
EXP_PER_DEV = N_EXP // N_DEV
SLOTS = EXP_PER_DEV * CAP


def _moe_pallas(x, gslot_row, gslot_col, expert_W):
    def body(x_ref, gsr_ref, gsc_ref, w_ref, out_ref, ygath_ref,
             send_sems, recv_sems):
        my = lax.axis_index("i")
        left = (my - 1) % N_DEV
        right = (my + 1) % N_DEV

        barrier = pltpu.get_barrier_semaphore()
        for nbr in (left, right):
            pl.semaphore_signal(
                barrier, inc=1,
                device_id=(nbr,), device_id_type=pl.DeviceIdType.MESH,
            )
        pl.semaphore_wait(barrier, 2)

        gsr = gsr_ref[0, :]
        xv = x_ref[:, :]

        for k in range(EXP_PER_DEV):
            base = (my * EXP_PER_DEV + k) * CAP
            slot_iota = lax.broadcasted_iota(jnp.int32, (CAP, N_TOK), 0)
            sel = (gsr[None, :] == slot_iota + base).astype(jnp.float32)
            xk = jnp.dot(sel, xv, preferred_element_type=jnp.float32)
            yk = jnp.dot(xk, w_ref[k], preferred_element_type=jnp.float32)
            ygath_ref[pl.ds(my * EXP_PER_DEV + k, 1)] = yk.astype(jnp.bfloat16)[None]

        gsc = gsc_ref[:, :]

        def combine(origin):
            for j in range(EXP_PER_DEV):
                e = origin * EXP_PER_DEV + j
                cap_iota = lax.broadcasted_iota(jnp.int32, (N_TOK, CAP), 1)
                p = (gsc == cap_iota + e * CAP).astype(jnp.bfloat16)
                ye = ygath_ref[pl.ds(e, 1)][0]
                out_ref[:, :] += jnp.dot(p, ye, preferred_element_type=jnp.float32)

        out_ref[:, :] = jnp.zeros((N_TOK, D_HID), jnp.float32)

        prev = None
        for h in range(N_DEV - 1):
            src = (my - h) % N_DEV
            rdma = pltpu.make_async_remote_copy(
                src_ref=ygath_ref.at[pl.ds(src * EXP_PER_DEV, EXP_PER_DEV)],
                dst_ref=ygath_ref.at[pl.ds(src * EXP_PER_DEV, EXP_PER_DEV)],
                send_sem=send_sems.at[h],
                recv_sem=recv_sems.at[h],
                device_id=(right,),
                device_id_type=pl.DeviceIdType.MESH,
            )
            rdma.start()
            if prev is not None:
                prev.wait_send()
            combine(src)
            rdma.wait_recv()
            prev = rdma
        prev.wait_send()
        combine((my - N_DEV + 1) % N_DEV)

    return pl.pallas_call(
        body,
        out_shape=jax.ShapeDtypeStruct((N_TOK, D_HID), jnp.float32),
        in_specs=[pl.BlockSpec(memory_space=pltpu.VMEM)] * 4,
        out_specs=pl.BlockSpec(memory_space=pltpu.VMEM),
        scratch_shapes=[
            pltpu.VMEM((N_EXP, CAP, D_HID), jnp.bfloat16),
            pltpu.SemaphoreType.DMA((N_DEV - 1,)),
            pltpu.SemaphoreType.DMA((N_DEV - 1,)),
        ],
        compiler_params=pltpu.CompilerParams(collective_id=0),
    )(x, gslot_row, gslot_col, expert_W)


def kernel(x, router_W, route_idx, expert_W):
    del router_W

    e_tok = route_idx[:, 0].astype(jnp.int32)
    onehot = e_tok[:, None] == jnp.arange(N_EXP, dtype=jnp.int32)[None, :]
    onehot = onehot.astype(jnp.int32)
    pos = jnp.cumsum(onehot, axis=0) - 1
    pos_tok = jnp.sum(pos * onehot, axis=1)
    keep = pos_tok < CAP
    gslot = jnp.where(keep, e_tok * CAP + pos_tok, N_EXP * CAP)

    return _moe_pallas(
        x, gslot.reshape(1, N_TOK), gslot.reshape(N_TOK, 1), expert_W
    )


# device time: 44629 ns/iter; 15.3761x vs baseline; 1.3148x over previous
import jax
import jax.numpy as jnp
from jax import lax
from jax.experimental import pallas as pl
from jax.experimental.pallas import tpu as pltpu

N_DEV = 4
N_TOK = 2048
D_MODEL = 512
D_HID = 1024
N_EXP = 16
CAP = 102
EXP_PER_DEV = N_EXP // N_DEV
SLOTS = EXP_PER_DEV * CAP


def _moe_pallas(x, gslot_row, gslot_col, expert_W):
    def body(x_ref, gsr_ref, gsc_ref, w_ref, out_ref, ygath_ref,
             sr_send, sr_recv, sl_send, sl_recv, sf_send, sf_recv):
        my = lax.axis_index("i")
        left = (my - 1) % N_DEV
        right = (my + 1) % N_DEV

        barrier = pltpu.get_barrier_semaphore()
        for nbr in (left, right):
            pl.semaphore_signal(
                barrier, inc=1,
                device_id=(nbr,), device_id_type=pl.DeviceIdType.MESH,
            )
        pl.semaphore_wait(barrier, 2)

        gsr = gsr_ref[0, :]
        gsc = gsc_ref[:, :]
        xv = x_ref[:, :]

        def block_rdma(origin_e, send_sems, recv_sems, k, target):
            blk = ygath_ref.at[pl.ds(origin_e + k, 1)]
            return pltpu.make_async_remote_copy(
                src_ref=blk, dst_ref=blk,
                send_sem=send_sems.at[k], recv_sem=recv_sems.at[k],
                device_id=(target,), device_id_type=pl.DeviceIdType.MESH,
            )

        send_r, send_l = [], []
        for k in range(EXP_PER_DEV):
            base = (my * EXP_PER_DEV + k) * CAP
            slot_iota = lax.broadcasted_iota(jnp.int32, (CAP, N_TOK), 0)
            sel = (gsr[None, :] == slot_iota + base).astype(jnp.float32)
            xk = jnp.dot(sel, xv, preferred_element_type=jnp.float32)
            yk = jnp.dot(xk, w_ref[k], preferred_element_type=jnp.float32)
            ygath_ref[pl.ds(my * EXP_PER_DEV + k, 1)] = yk.astype(jnp.bfloat16)[None]
            r = block_rdma(my * EXP_PER_DEV, sr_send, sr_recv, k, right)
            l = block_rdma(my * EXP_PER_DEV, sl_send, sl_recv, k, left)
            r.start()
            l.start()
            send_r.append(r)
            send_l.append(l)

        first = [True]

        def combine(e):
            cap_iota = lax.broadcasted_iota(jnp.int32, (N_TOK, CAP), 1)
            p = (gsc == cap_iota + e * CAP).astype(jnp.bfloat16)
            ye = ygath_ref[pl.ds(e, 1)][0]
            contrib = jnp.dot(p, ye, preferred_element_type=jnp.float32)
            if first[0]:
                out_ref[:, :] = contrib
                first[0] = False
            else:
                out_ref[:, :] += contrib

        for k in range(EXP_PER_DEV):
            combine(my * EXP_PER_DEV + k)

        fwd = []
        for k in range(EXP_PER_DEV):
            block_rdma(left * EXP_PER_DEV, sr_send, sr_recv, k, right).wait_recv()
            f = block_rdma(left * EXP_PER_DEV, sf_send, sf_recv, k, right)
            f.start()
            fwd.append(f)
        for k in range(EXP_PER_DEV):
            combine(left * EXP_PER_DEV + k)

        for k in range(EXP_PER_DEV):
            block_rdma(right * EXP_PER_DEV, sl_send, sl_recv, k, left).wait_recv()
        for k in range(EXP_PER_DEV):
            combine(right * EXP_PER_DEV + k)

        diag = (my + 2) % N_DEV
        for k in range(EXP_PER_DEV):
            block_rdma(diag * EXP_PER_DEV, sf_send, sf_recv, k, right).wait_recv()
        for k in range(EXP_PER_DEV):
            combine(diag * EXP_PER_DEV + k)

        for d in send_r + send_l + fwd:
            d.wait_send()

    return pl.pallas_call(
        body,
        out_shape=jax.ShapeDtypeStruct((N_TOK, D_HID), jnp.float32),
        in_specs=[pl.BlockSpec(memory_space=pltpu.VMEM)] * 4,
        out_specs=pl.BlockSpec(memory_space=pltpu.VMEM),
        scratch_shapes=[
            pltpu.VMEM((N_EXP, CAP, D_HID), jnp.bfloat16),
            pltpu.SemaphoreType.DMA((EXP_PER_DEV,)),
            pltpu.SemaphoreType.DMA((EXP_PER_DEV,)),
            pltpu.SemaphoreType.DMA((EXP_PER_DEV,)),
            pltpu.SemaphoreType.DMA((EXP_PER_DEV,)),
            pltpu.SemaphoreType.DMA((EXP_PER_DEV,)),
            pltpu.SemaphoreType.DMA((EXP_PER_DEV,)),
        ],
        compiler_params=pltpu.CompilerParams(collective_id=0),
    )(x, gslot_row, gslot_col, expert_W)


def kernel(x, router_W, route_idx, expert_W):
    del router_W

    e_tok = route_idx[:, 0].astype(jnp.int32)
    onehot = e_tok[:, None] == jnp.arange(N_EXP, dtype=jnp.int32)[None, :]
    onehot = onehot.astype(jnp.int32)
    pos = jnp.cumsum(onehot, axis=0) - 1
    pos_tok = jnp.sum(pos * onehot, axis=1)
    keep = pos_tok < CAP
    gslot = jnp.where(keep, e_tok * CAP + pos_tok, N_EXP * CAP)

    return _moe_pallas(
        x, gslot.reshape(1, N_TOK), gslot.reshape(N_TOK, 1), expert_W
    )


# device time: 44607 ns/iter; 15.3837x vs baseline; 1.0005x over previous
import jax
import jax.numpy as jnp
from jax import lax
from jax.experimental import pallas as pl
from jax.experimental.pallas import tpu as pltpu

N_DEV = 4
N_TOK = 2048
D_MODEL = 512
D_HID = 1024
N_EXP = 16
CAP = 102
EXP_PER_DEV = N_EXP // N_DEV
SLOTS = EXP_PER_DEV * CAP


def _moe_pallas(x, gslot_row, gslot_col, expert_W):
    def body(x_ref, gsr_ref, gsc_ref, w_ref, out_ref, ygath_ref,
             sr_send, sr_recv, sl_send, sl_recv, sf_send, sf_recv):
        my = lax.axis_index("i")
        left = (my - 1) % N_DEV
        right = (my + 1) % N_DEV

        barrier = pltpu.get_barrier_semaphore()
        for nbr in (left, right):
            pl.semaphore_signal(
                barrier, inc=1,
                device_id=(nbr,), device_id_type=pl.DeviceIdType.MESH,
            )
        pl.semaphore_wait(barrier, 2)

        gsr = gsr_ref[0, :]
        gsc = gsc_ref[:, :]
        xv = x_ref[:, :].astype(jnp.bfloat16)

        def block_rdma(origin_e, send_sems, recv_sems, k, target):
            blk = ygath_ref.at[pl.ds(origin_e + k, 1)]
            return pltpu.make_async_remote_copy(
                src_ref=blk, dst_ref=blk,
                send_sem=send_sems.at[k], recv_sem=recv_sems.at[k],
                device_id=(target,), device_id_type=pl.DeviceIdType.MESH,
            )

        send_r, send_l = [], []
        for k in range(EXP_PER_DEV):
            base = (my * EXP_PER_DEV + k) * CAP
            slot_iota = lax.broadcasted_iota(jnp.int32, (CAP, N_TOK), 0)
            sel = (gsr[None, :] == slot_iota + base).astype(jnp.bfloat16)
            xk = jnp.dot(sel, xv,
                         preferred_element_type=jnp.float32).astype(jnp.bfloat16)
            yk = jnp.dot(xk, w_ref[k].astype(jnp.bfloat16),
                         preferred_element_type=jnp.float32)
            ygath_ref[pl.ds(my * EXP_PER_DEV + k, 1)] = yk.astype(jnp.bfloat16)[None]
            r = block_rdma(my * EXP_PER_DEV, sr_send, sr_recv, k, right)
            l = block_rdma(my * EXP_PER_DEV, sl_send, sl_recv, k, left)
            r.start()
            l.start()
            send_r.append(r)
            send_l.append(l)

        first = [True]

        def combine(e):
            cap_iota = lax.broadcasted_iota(jnp.int32, (N_TOK, CAP), 1)
            p = (gsc == cap_iota + e * CAP).astype(jnp.bfloat16)
            ye = ygath_ref[pl.ds(e, 1)][0]
            contrib = jnp.dot(p, ye, preferred_element_type=jnp.float32)
            if first[0]:
                out_ref[:, :] = contrib
                first[0] = False
            else:
                out_ref[:, :] += contrib

        for k in range(EXP_PER_DEV):
            combine(my * EXP_PER_DEV + k)

        fwd = []
        for k in range(EXP_PER_DEV):
            block_rdma(left * EXP_PER_DEV, sr_send, sr_recv, k, right).wait_recv()
            f = block_rdma(left * EXP_PER_DEV, sf_send, sf_recv, k, right)
            f.start()
            fwd.append(f)
        for k in range(EXP_PER_DEV):
            combine(left * EXP_PER_DEV + k)

        for k in range(EXP_PER_DEV):
            block_rdma(right * EXP_PER_DEV, sl_send, sl_recv, k, left).wait_recv()
        for k in range(EXP_PER_DEV):
            combine(right * EXP_PER_DEV + k)

        diag = (my + 2) % N_DEV
        for k in range(EXP_PER_DEV):
            block_rdma(diag * EXP_PER_DEV, sf_send, sf_recv, k, right).wait_recv()
        for k in range(EXP_PER_DEV):
            combine(diag * EXP_PER_DEV + k)

        for d in send_r + send_l + fwd:
            d.wait_send()

    return pl.pallas_call(
        body,
        out_shape=jax.ShapeDtypeStruct((N_TOK, D_HID), jnp.float32),
        in_specs=[pl.BlockSpec(memory_space=pltpu.VMEM)] * 4,
        out_specs=pl.BlockSpec(memory_space=pltpu.VMEM),
        scratch_shapes=[
            pltpu.VMEM((N_EXP, CAP, D_HID), jnp.bfloat16),
            pltpu.SemaphoreType.DMA((EXP_PER_DEV,)),
            pltpu.SemaphoreType.DMA((EXP_PER_DEV,)),
            pltpu.SemaphoreType.DMA((EXP_PER_DEV,)),
            pltpu.SemaphoreType.DMA((EXP_PER_DEV,)),
            pltpu.SemaphoreType.DMA((EXP_PER_DEV,)),
            pltpu.SemaphoreType.DMA((EXP_PER_DEV,)),
        ],
        compiler_params=pltpu.CompilerParams(collective_id=0),
    )(x, gslot_row, gslot_col, expert_W)


def kernel(x, router_W, route_idx, expert_W):
    del router_W

    e_tok = route_idx[:, 0].astype(jnp.int32)
    onehot = e_tok[:, None] == jnp.arange(N_EXP, dtype=jnp.int32)[None, :]
    onehot = onehot.astype(jnp.int32)
    pos = jnp.cumsum(onehot, axis=0) - 1
    pos_tok = jnp.sum(pos * onehot, axis=1)
    keep = pos_tok < CAP
    gslot = jnp.where(keep, e_tok * CAP + pos_tok, N_EXP * CAP)

    return _moe_pallas(
        x, gslot.reshape(1, N_TOK), gslot.reshape(N_TOK, 1), expert_W
    )


# device time: 40643 ns/iter; 16.8841x vs baseline; 1.0975x over previous
import jax
import jax.numpy as jnp
from jax import lax
from jax.experimental import pallas as pl
from jax.experimental.pallas import tpu as pltpu

N_DEV = 4
N_TOK = 2048
D_MODEL = 512
D_HID = 1024
N_EXP = 16
CAP = 102
EXP_PER_DEV = N_EXP // N_DEV
SLOTS = EXP_PER_DEV * CAP
SENTINEL = N_EXP * CAP


def _gslot_cols(rid):
    lane = lax.broadcasted_iota(jnp.int32, (N_TOK, N_EXP), 1)
    oh = (rid == lane).astype(jnp.int32)
    c, s = oh, 1
    while s < N_TOK:
        c = c + jnp.concatenate(
            [jnp.zeros((s, N_EXP), jnp.int32), c[: N_TOK - s]], axis=0)
        s *= 2
    pos = jnp.sum((c - 1) * oh, axis=1, keepdims=True)
    return jnp.where(pos < CAP, rid * CAP + pos, SENTINEL)


def _gslot_rows(rid_t):
    sub = lax.broadcasted_iota(jnp.int32, (N_EXP, N_TOK), 0)
    oh = (rid_t == sub).astype(jnp.int32)
    c, s = oh, 1
    while s < N_TOK:
        c = c + jnp.concatenate(
            [jnp.zeros((N_EXP, s), jnp.int32), c[:, : N_TOK - s]], axis=1)
        s *= 2
    pos = jnp.sum((c - 1) * oh, axis=0, keepdims=True)
    return jnp.where(pos < CAP, rid_t * CAP + pos, SENTINEL)


def _moe_pallas(x, rid, rid_t, expert_W):
    def body(x_ref, rid_ref, ridt_ref, w_ref, out_ref, ygath_ref,
             sr_send, sr_recv, sl_send, sl_recv, sf_send, sf_recv):
        my = lax.axis_index("i")
        left = (my - 1) % N_DEV
        right = (my + 1) % N_DEV

        barrier = pltpu.get_barrier_semaphore()
        for nbr in (left, right):
            pl.semaphore_signal(
                barrier, inc=1,
                device_id=(nbr,), device_id_type=pl.DeviceIdType.MESH,
            )
        pl.semaphore_wait(barrier, 2)

        gsr = _gslot_rows(ridt_ref[:, :])[0, :]
        gsc = _gslot_cols(rid_ref[:, :])
        xv = x_ref[:, :].astype(jnp.bfloat16)

        def block_rdma(origin_e, send_sems, recv_sems, k, target):
            blk = ygath_ref.at[pl.ds(origin_e + k, 1)]
            return pltpu.make_async_remote_copy(
                src_ref=blk, dst_ref=blk,
                send_sem=send_sems.at[k], recv_sem=recv_sems.at[k],
                device_id=(target,), device_id_type=pl.DeviceIdType.MESH,
            )

        send_r, send_l = [], []
        for k in range(EXP_PER_DEV):
            base = (my * EXP_PER_DEV + k) * CAP
            slot_iota = lax.broadcasted_iota(jnp.int32, (CAP, N_TOK), 0)
            sel = (gsr[None, :] == slot_iota + base).astype(jnp.bfloat16)
            xk = jnp.dot(sel, xv,
                         preferred_element_type=jnp.float32).astype(jnp.bfloat16)
            yk = jnp.dot(xk, w_ref[k].astype(jnp.bfloat16),
                         preferred_element_type=jnp.float32)
            ygath_ref[pl.ds(my * EXP_PER_DEV + k, 1)] = yk.astype(jnp.bfloat16)[None]
            r = block_rdma(my * EXP_PER_DEV, sr_send, sr_recv, k, right)
            l = block_rdma(my * EXP_PER_DEV, sl_send, sl_recv, k, left)
            r.start()
            l.start()
            send_r.append(r)
            send_l.append(l)

        first = [True]

        def combine(e):
            cap_iota = lax.broadcasted_iota(jnp.int32, (N_TOK, CAP), 1)
            p = (gsc == cap_iota + e * CAP).astype(jnp.bfloat16)
            ye = ygath_ref[pl.ds(e, 1)][0]
            contrib = jnp.dot(p, ye, preferred_element_type=jnp.float32)
            if first[0]:
                out_ref[:, :] = contrib
                first[0] = False
            else:
                out_ref[:, :] += contrib

        for k in range(EXP_PER_DEV):
            combine(my * EXP_PER_DEV + k)

        fwd = []
        for k in range(EXP_PER_DEV):
            block_rdma(left * EXP_PER_DEV, sr_send, sr_recv, k, right).wait_recv()
            f = block_rdma(left * EXP_PER_DEV, sf_send, sf_recv, k, right)
            f.start()
            fwd.append(f)
        for k in range(EXP_PER_DEV):
            combine(left * EXP_PER_DEV + k)

        for k in range(EXP_PER_DEV):
            block_rdma(right * EXP_PER_DEV, sl_send, sl_recv, k, left).wait_recv()
        for k in range(EXP_PER_DEV):
            combine(right * EXP_PER_DEV + k)

        diag = (my + 2) % N_DEV
        for k in range(EXP_PER_DEV):
            block_rdma(diag * EXP_PER_DEV, sf_send, sf_recv, k, right).wait_recv()
        for k in range(EXP_PER_DEV):
            combine(diag * EXP_PER_DEV + k)

        for d in send_r + send_l + fwd:
            d.wait_send()

    return pl.pallas_call(
        body,
        out_shape=jax.ShapeDtypeStruct((N_TOK, D_HID), jnp.float32),
        in_specs=[pl.BlockSpec(memory_space=pltpu.VMEM)] * 4,
        out_specs=pl.BlockSpec(memory_space=pltpu.VMEM),
        scratch_shapes=[
            pltpu.VMEM((N_EXP, CAP, D_HID), jnp.bfloat16),
            pltpu.SemaphoreType.DMA((EXP_PER_DEV,)),
            pltpu.SemaphoreType.DMA((EXP_PER_DEV,)),
            pltpu.SemaphoreType.DMA((EXP_PER_DEV,)),
            pltpu.SemaphoreType.DMA((EXP_PER_DEV,)),
            pltpu.SemaphoreType.DMA((EXP_PER_DEV,)),
            pltpu.SemaphoreType.DMA((EXP_PER_DEV,)),
        ],
        compiler_params=pltpu.CompilerParams(collective_id=0),
    )(x, rid, rid_t, expert_W)


def kernel(x, router_W, route_idx, expert_W):
    del router_W
    rid = route_idx.astype(jnp.int32)
    return _moe_pallas(x, rid, rid.T, expert_W)
